# baseline (speedup 1.0000x reference)
import math

import jax
import jax.numpy as jnp
from jax import lax
from jax.experimental import pallas as pl
from jax.experimental.pallas import tpu as pltpu
from jax.experimental.pallas import tpu_sc as plsc

_SINK_SIZE = 16
_SLIDING_WINDOW = 4080
_NUM_KV_HEADS = 8
_HEAD_SIZE = 128
_BLOCK_SIZE = 16
_X = 8
_NUM_BLOCKS = 2048
_BATCH = 64
_ROPE_BASE = 10000.0

_CACHE_SIZE = float(_SLIDING_WINDOW + _SINK_SIZE)
_HALF = _HEAD_SIZE // 2
_NROWS = 16384
_NC = 2
_NS = 16
_NW = _NC * _NS
_NGROUPS = _NUM_KV_HEADS * (_HEAD_SIZE // _X // 2)
_GPW = _NGROUPS // _NW
_CR = 4
_CPG = 128 // _CR
_CPW = _GPW * _CPG
_NSLOT = 3


def _tables_body(btc_ref, posc_ref, cos_ref, sin_ref):
    btc = btc_ref[...]
    posc = posc_ref[...]

    iota_b = lax.broadcasted_iota(jnp.int32, (_BATCH, _NUM_BLOCKS), 1)
    eq = btc == iota_b
    theta = jnp.maximum(posc.astype(jnp.float32) - _CACHE_SIZE, 0.0)
    masked = jnp.where(eq, jnp.broadcast_to(theta, (_BATCH, _NUM_BLOCKS)), 0.0)
    angle = jnp.sum(masked, axis=0, keepdims=True)

    fcol = lax.broadcasted_iota(jnp.int32, (_HALF, 1), 0).astype(jnp.float32)
    inv_freq = jnp.exp(fcol * (-2.0 * math.log(_ROPE_BASE) / _HEAD_SIZE))
    ang = inv_freq * angle
    cos_ref[...] = jnp.cos(ang)
    sin_ref[...] = jnp.sin(ang)


def _make_tables(interpret=False):
    return pl.pallas_call(
        _tables_body,
        out_shape=(
            jax.ShapeDtypeStruct((_HALF, _NUM_BLOCKS), jnp.float32),
            jax.ShapeDtypeStruct((_HALF, _NUM_BLOCKS), jnp.float32),
        ),
        interpret=interpret,
    )


def _sc_body(in_hbm, c_hbm, s_hbm, out_hbm,
             bufl, bufh, c_v, s_v, inl_sems, inh_sems, outl_sems, outh_sems):
    cid = lax.axis_index("c")
    sid = lax.axis_index("s")
    wid = sid * _NC + cid

    def rows_of(k):
        g = wid * _GPW + k // _CPG
        kc = k % _CPG
        h = g // 8
        dx = g - h * 8
        low = h * 2048 + dx * 128 + kc * _CR
        return low, dx, kc

    def start_in(k):
        low, _, _ = rows_of(k)
        slot = k % _NSLOT
        pltpu.make_async_copy(
            in_hbm.at[pl.ds(low, _CR)],
            bufl.at[pl.ds(slot * _CR, _CR)],
            inl_sems.at[slot]).start()
        pltpu.make_async_copy(
            in_hbm.at[pl.ds(low + 1024, _CR)],
            bufh.at[pl.ds(slot * _CR, _CR)],
            inh_sems.at[slot]).start()

    def wait_in(k):
        low, _, _ = rows_of(k)
        slot = k % _NSLOT
        pltpu.make_async_copy(
            in_hbm.at[pl.ds(low, _CR)],
            bufl.at[pl.ds(slot * _CR, _CR)],
            inl_sems.at[slot]).wait()
        pltpu.make_async_copy(
            in_hbm.at[pl.ds(low + 1024, _CR)],
            bufh.at[pl.ds(slot * _CR, _CR)],
            inh_sems.at[slot]).wait()

    def start_out(k):
        low, _, _ = rows_of(k)
        slot = k % _NSLOT
        pltpu.make_async_copy(
            bufl.at[pl.ds(slot * _CR, _CR)],
            out_hbm.at[pl.ds(low, _CR)],
            outl_sems.at[slot]).start()
        pltpu.make_async_copy(
            bufh.at[pl.ds(slot * _CR, _CR)],
            out_hbm.at[pl.ds(low + 1024, _CR)],
            outh_sems.at[slot]).start()

    def wait_out(k):
        low, _, _ = rows_of(k)
        slot = k % _NSLOT
        pltpu.make_async_copy(
            bufl.at[pl.ds(slot * _CR, _CR)],
            out_hbm.at[pl.ds(low, _CR)],
            outl_sems.at[slot]).wait()
        pltpu.make_async_copy(
            bufh.at[pl.ds(slot * _CR, _CR)],
            out_hbm.at[pl.ds(low + 1024, _CR)],
            outh_sems.at[slot]).wait()

    start_in(0)
    start_in(1)

    def step(k, carry):
        _, dx, kc = rows_of(k)
        slot = k % _NSLOT

        @pl.when(kc == 0)
        def _():
            pltpu.sync_copy(c_hbm.at[pl.ds(dx * 8, 8)], c_v)
            pltpu.sync_copy(s_hbm.at[pl.ds(dx * 8, 8)], s_v)

        wait_in(k)
        xb = (k % 2) * _CR

        def comp(v4, carry2):
            for u in range(4):
                o = (v4 * 4 + u) * 16
                for i in range(_CR):
                    c = c_v[xb + i, pl.ds(o, 16)]
                    s = s_v[xb + i, pl.ds(o, 16)]
                    k1 = bufl[slot * _CR + i, pl.ds(o, 16)]
                    k2 = bufh[slot * _CR + i, pl.ds(o, 16)]
                    bufl[slot * _CR + i, pl.ds(o, 16)] = k1 * c - k2 * s
                    bufh[slot * _CR + i, pl.ds(o, 16)] = k2 * c + k1 * s
            return carry2

        lax.fori_loop(0, _NUM_BLOCKS // 64, comp, 0)
        start_out(k)

        @pl.when(k >= 1)
        def _():
            wait_out(k - 1)

        @pl.when(k + 2 < _CPW)
        def _():
            start_in(k + 2)

        return carry

    lax.fori_loop(0, _CPW, step, 0)
    wait_out(_CPW - 1)


def _make_sc_apply(interpret=False):
    mesh = plsc.VectorSubcoreMesh(
        core_axis_name="c", subcore_axis_name="s",
        num_cores=_NC, num_subcores=_NS)
    return pl.kernel(
        _sc_body,
        out_type=jax.ShapeDtypeStruct((_NROWS, _NUM_BLOCKS), jnp.float32),
        mesh=mesh,
        compiler_params=pltpu.CompilerParams(
            needs_layout_passes=False, use_tc_tiling_on_sc=True),
        scratch_types=[
            pltpu.VMEM((_NSLOT * _CR, _NUM_BLOCKS), jnp.float32),
            pltpu.VMEM((_NSLOT * _CR, _NUM_BLOCKS), jnp.float32),
            pltpu.VMEM((8, _NUM_BLOCKS), jnp.float32),
            pltpu.VMEM((8, _NUM_BLOCKS), jnp.float32),
            pltpu.SemaphoreType.DMA((_NSLOT,)),
            pltpu.SemaphoreType.DMA((_NSLOT,)),
            pltpu.SemaphoreType.DMA((_NSLOT,)),
            pltpu.SemaphoreType.DMA((_NSLOT,)),
        ],
        interpret=interpret,
    )


def _kernel_impl(key_cache, block_tables, context_lens, positions,
                 interpret=False):
    del context_lens
    m = jnp.transpose(key_cache, (1, 2, 3, 4, 0)).reshape(_NROWS, _NUM_BLOCKS)
    btc = block_tables[:, :1]
    posc = positions.reshape(_BATCH, 1)
    cos_t, sin_t = _make_tables(interpret)(btc, posc)
    out = _make_sc_apply(interpret)(m, cos_t, sin_t)
    out5 = out.reshape(_NUM_KV_HEADS, _HEAD_SIZE // _X, _BLOCK_SIZE, _X,
                       _NUM_BLOCKS)
    return jnp.transpose(out5, (4, 0, 1, 2, 3))


def kernel(key_cache, block_tables, context_lens, positions):
    return _kernel_impl(key_cache, block_tables, context_lens, positions)
